# Initial kernel scaffold; baseline (speedup 1.0000x reference)
import jax
import jax.numpy as jnp
from jax.experimental import pallas as pl


def kernel(positions, node_attrs, edge_index, shifts, batch, ptr, W_emb, W_at, b_at, R1, R2, R3, R4, W_up, W_msg, W_sc, U, Wr0, Wr1a, Wr1b):
    raise NotImplementedError("write your pallas kernel here")


# speedup vs baseline: 1.2170x; 1.2170x over previous
import functools
import math

import jax
import jax.numpy as jnp
from jax import lax
from jax.experimental import pallas as pl
from jax.experimental.pallas import tpu as pltpu
from jax.experimental.pallas import tpu_sc as plsc

N = 10000
E = 160000
NE = 10
C = 32
NB = 8
NG = 16
SH = 16
RMAX = 5.0
AVG = 16.0
EPS = 1e-08

N_PAD = 10240
E_PAD = 163840
NW = 32
EPW = E_PAD // NW
CH = 128
NCH = EPW // CH
SUP = 8
SUPE = SUP * CH
NSUP = NCH // SUP
EB = 1024
EGRID = E_PAD // EB

_S3 = math.sqrt(3.0)
_S5 = math.sqrt(5.0)
_S7 = math.sqrt(7.0)
_S15 = math.sqrt(15.0)
_S42 = math.sqrt(42.0)
_S70 = math.sqrt(70.0)
_S105 = math.sqrt(105.0)

@functools.cache
def _mesh():
    return plsc.VectorSubcoreMesh(core_axis_name="c", subcore_axis_name="s",
                                  num_cores=2, num_subcores=16)


_f32 = jnp.float32


NBLK = 1024
NGRID = N_PAD // NBLK


def _onehot_energy(bat, ne):
    oh = (bat == lax.broadcasted_iota(jnp.int32, (NBLK, NG), 1)).astype(_f32)
    return jnp.sum(oh * ne, axis=0, keepdims=True)


def _acc_energy(e_ref, contrib):
    @pl.when(pl.program_id(0) == 0)
    def _():
        e_ref[...] = jnp.zeros_like(e_ref)

    e_ref[...] += contrib


def _embed_body(na_ref, bat_ref, wemb_ref, wat_ref, bat_b_ref, wup_ref,
                nf_ref, h_ref, e_ref):
    na = na_ref[...]
    nf = jnp.dot(na, wemb_ref[...], preferred_element_type=_f32)
    nf_ref[...] = nf
    h_ref[...] = jnp.dot(nf, wup_ref[...], preferred_element_type=_f32)
    ne = jnp.dot(na, wat_ref[...], preferred_element_type=_f32) + bat_b_ref[0, 0]
    _acc_energy(e_ref, _onehot_energy(bat_ref[...], ne))


def _nb_spec(w):
    return pl.BlockSpec((NBLK, w), lambda i: (i, 0))


def _w_spec(*dims):
    nd = len(dims)
    return pl.BlockSpec(dims, lambda i: (0,) * nd)


def _tc_embed(na_p, bat_p, W_emb, W_at, b_at2, Wup0):
    return pl.pallas_call(
        _embed_body,
        grid=(NGRID,),
        in_specs=[
            _nb_spec(NE), _nb_spec(1), _w_spec(NE, C), _w_spec(NE, 1),
            _w_spec(1, 1), _w_spec(C, C),
        ],
        out_specs=[_nb_spec(C), _nb_spec(C), _w_spec(1, NG)],
        out_shape=[
            jax.ShapeDtypeStruct((N_PAD, C), _f32),
            jax.ShapeDtypeStruct((N_PAD, C), _f32),
            jax.ShapeDtypeStruct((1, NG), _f32),
        ],
    )(na_p, bat_p, W_emb, W_at, b_at2, Wup0)


def _radial_mlp(ef, r1, r2, r3, r4):
    a = jax.nn.silu(jnp.dot(ef, r1, preferred_element_type=_f32))
    a = jax.nn.silu(jnp.dot(a, r2, preferred_element_type=_f32))
    a = jax.nn.silu(jnp.dot(a, r3, preferred_element_type=_f32))
    return jnp.dot(a, r4, preferred_element_type=_f32)


def _edge_msg(he, ysph, wm):
    msg = (he[:, :, None] * ysph[:, None, :]).reshape(EB, C * SH)
    return jnp.dot(msg, wm, preferred_element_type=_f32)


def _edge0_body(ps_ref, pr_ref, sh_ref, hs_ref, r1_ref, r2_ref, r3_ref,
                r4_ref, wm_ref, z_ref, y_ref, ef_ref):
    i = pl.program_id(0)
    vec = pr_ref[...] - ps_ref[...] + sh_ref[...]
    ln = jnp.sqrt(jnp.sum(vec * vec, axis=1, keepdims=True))
    lc = jnp.maximum(ln, EPS)
    u = vec / lc
    x = u[:, 0:1]
    y = u[:, 1:2]
    z = u[:, 2:3]
    x2 = x * x
    y2 = y * y
    z2 = z * z
    ysph = jnp.concatenate([
        jnp.ones_like(x),
        _S3 * x, _S3 * y, _S3 * z,
        _S15 * x * y, _S15 * y * z, 0.5 * _S5 * (3.0 * z2 - 1.0),
        _S15 * x * z, 0.5 * _S15 * (x2 - y2),
        0.25 * _S70 * y * (3.0 * x2 - y2), _S105 * x * y * z,
        0.25 * _S42 * y * (5.0 * z2 - 1.0),
        0.5 * _S7 * z * (5.0 * z2 - 3.0),
        0.25 * _S42 * x * (5.0 * z2 - 1.0),
        0.5 * _S105 * z * (x2 - y2),
        0.25 * _S70 * x * (x2 - 3.0 * y2),
    ], axis=1)
    y_ref[...] = ysph
    nvec = lax.broadcasted_iota(jnp.int32, (1, NB), 1).astype(_f32) + 1.0
    bes = math.sqrt(2.0 / RMAX) * jnp.sin(nvec * (math.pi / RMAX) * lc) / lc
    xx = lc * (1.0 / RMAX)
    xp2 = xx * xx
    xp4 = xp2 * xp2
    xp5 = xp4 * xx
    env = 1.0 - 21.0 * xp5 + 35.0 * xp5 * xx - 15.0 * xp5 * xp2
    env = env * (xx < 1.0).astype(_f32)
    ef = bes * env
    eid = i * EB + lax.broadcasted_iota(jnp.int32, (EB, 1), 0)
    ef = jnp.where(eid < E, ef, 0.0)
    ef_ref[...] = ef
    r = _radial_mlp(ef, r1_ref[...], r2_ref[...], r3_ref[...], r4_ref[...])
    he = hs_ref[...] * r
    z_ref[...] = _edge_msg(he, ysph, wm_ref[...])


def _tc_edge0(ps, pr, sh4, hs, r1, r2, r3, r4, wm):
    eb_spec = lambda w: pl.BlockSpec((EB, w), lambda i: (i, 0))
    wspec = lambda a, b: pl.BlockSpec((a, b), lambda i: (0, 0))
    return pl.pallas_call(
        _edge0_body,
        grid=(EGRID,),
        in_specs=[
            eb_spec(4), eb_spec(4), eb_spec(4), eb_spec(C),
            wspec(NB, 64), wspec(64, 64), wspec(64, 64), wspec(64, C),
            wspec(C * SH, C),
        ],
        out_specs=[eb_spec(C), eb_spec(SH), eb_spec(NB)],
        out_shape=[
            jax.ShapeDtypeStruct((E_PAD, C), _f32),
            jax.ShapeDtypeStruct((E_PAD, SH), _f32),
            jax.ShapeDtypeStruct((E_PAD, NB), _f32),
        ],
    )(ps, pr, sh4, hs, r1, r2, r3, r4, wm)


def _edge1_body(y_ref, ef_ref, hs_ref, r1_ref, r2_ref, r3_ref, r4_ref,
                wm_ref, z_ref):
    r = _radial_mlp(ef_ref[...], r1_ref[...], r2_ref[...], r3_ref[...],
                    r4_ref[...])
    he = hs_ref[...] * r
    z_ref[...] = _edge_msg(he, y_ref[...], wm_ref[...])


def _tc_edge1(ysph, ef, hs, r1, r2, r3, r4, wm):
    eb_spec = lambda w: pl.BlockSpec((EB, w), lambda i: (i, 0))
    wspec = lambda a, b: pl.BlockSpec((a, b), lambda i: (0, 0))
    return pl.pallas_call(
        _edge1_body,
        grid=(EGRID,),
        in_specs=[
            eb_spec(SH), eb_spec(NB), eb_spec(C),
            wspec(NB, 64), wspec(64, 64), wspec(64, 64), wspec(64, C),
            wspec(C * SH, C),
        ],
        out_specs=[eb_spec(C)],
        out_shape=[jax.ShapeDtypeStruct((E_PAD, C), _f32)],
    )(ysph, ef, hs, r1, r2, r3, r4, wm)[0]


def _node_update(mp, na, nf, wsc, uu):
    m = (mp[0] + mp[1]) * (1.0 / AVG)
    sc = jnp.zeros((NBLK, C), _f32)
    for k in range(NE):
        sc = sc + na[:, k:k + 1] * jnp.dot(nf, wsc[k],
                                           preferred_element_type=_f32)
    m2 = m * m
    m3 = m2 * m
    return (sc
            + jnp.dot(m, uu[0], preferred_element_type=_f32)
            + jnp.dot(m2, uu[1], preferred_element_type=_f32)
            + jnp.dot(m3, uu[2], preferred_element_type=_f32))


def _update0_body(mp_ref, na_ref, nf_ref, bat_ref, wsc_ref, u_ref, wr0_ref,
                  wup_ref, nfo_ref, h_ref, e_ref):
    nfo = _node_update(mp_ref[...], na_ref[...], nf_ref[...], wsc_ref[...],
                       u_ref[...])
    nfo_ref[...] = nfo
    h_ref[...] = jnp.dot(nfo, wup_ref[...], preferred_element_type=_f32)
    ne = jnp.dot(nfo, wr0_ref[...], preferred_element_type=_f32)
    _acc_energy(e_ref, _onehot_energy(bat_ref[...], ne))


def _mp_spec():
    return pl.BlockSpec((2, NBLK, C), lambda i: (0, i, 0))


def _tc_update0(mp, na_p, nf, bat_p, wsc, uu, wr0, wup1):
    return pl.pallas_call(
        _update0_body,
        grid=(NGRID,),
        in_specs=[
            _mp_spec(), _nb_spec(NE), _nb_spec(C), _nb_spec(1),
            _w_spec(NE, C, C), _w_spec(3, C, C), _w_spec(C, 1),
            _w_spec(C, C),
        ],
        out_specs=[_nb_spec(C), _nb_spec(C), _w_spec(1, NG)],
        out_shape=[
            jax.ShapeDtypeStruct((N_PAD, C), _f32),
            jax.ShapeDtypeStruct((N_PAD, C), _f32),
            jax.ShapeDtypeStruct((1, NG), _f32),
        ],
    )(mp, na_p, nf, bat_p, wsc, uu, wr0, wup1)


def _update1_body(mp_ref, na_ref, nf_ref, bat_ref, wsc_ref, u_ref, wr1a_ref,
                  wr1b_ref, e0_ref, e1_ref, e_ref):
    nfo = _node_update(mp_ref[...], na_ref[...], nf_ref[...], wsc_ref[...],
                       u_ref[...])
    hid = jax.nn.silu(jnp.dot(nfo, wr1a_ref[...], preferred_element_type=_f32))
    ne = jnp.dot(hid, wr1b_ref[...], preferred_element_type=_f32)
    contrib = _onehot_energy(bat_ref[...], ne)

    @pl.when(pl.program_id(0) == 0)
    def _():
        e_ref[...] = e0_ref[...] + e1_ref[...]

    e_ref[...] += contrib


def _tc_update1(mp, na_p, nf, bat_p, wsc, uu, wr1a, wr1b, e0, e1):
    return pl.pallas_call(
        _update1_body,
        grid=(NGRID,),
        in_specs=[
            _mp_spec(), _nb_spec(NE), _nb_spec(C), _nb_spec(1),
            _w_spec(NE, C, C), _w_spec(3, C, C), _w_spec(C, SH),
            _w_spec(SH, 1), _w_spec(1, NG), _w_spec(1, NG),
        ],
        out_specs=_w_spec(1, NG),
        out_shape=jax.ShapeDtypeStruct((1, NG), _f32),
    )(mp, na_p, nf, bat_p, wsc, uu, wr1a, wr1b, e0, e1)


def _sc_gather3(pos4, h0, send2d, recv2d):
    @functools.partial(
        pl.kernel,
        out_type=[
            jax.ShapeDtypeStruct((E_PAD, 4), _f32),
            jax.ShapeDtypeStruct((E_PAD, 4), _f32),
            jax.ShapeDtypeStruct((E_PAD, C), _f32),
        ],
        mesh=_mesh(),
        compiler_params=pltpu.CompilerParams(use_tc_tiling_on_sc=False),
        scratch_types=[
            pltpu.VMEM((NCH, CH), jnp.int32),
            pltpu.VMEM((NCH, CH), jnp.int32),
            pltpu.VMEM((SUPE, 4), _f32),
            pltpu.VMEM((SUPE, 4), _f32),
            pltpu.VMEM((SUPE, C), _f32),
            pltpu.SemaphoreType.DMA,
        ],
    )
    def k(pos_hbm, h_hbm, sidx_hbm, ridx_hbm, ps_out, pr_out, hs_out,
          sidx_v, ridx_v, bs_v, br_v, bh_v, sem):
        wid = lax.axis_index("c") * 16 + lax.axis_index("s")
        pltpu.sync_copy(sidx_hbm.at[pl.ds(wid * NCH, NCH)], sidx_v)
        pltpu.sync_copy(ridx_hbm.at[pl.ds(wid * NCH, NCH)], ridx_v)

        def sup_body(si, carry):
            base = wid * EPW + si * SUPE
            descs = []
            for j in range(SUP):
                row = si * SUP + j
                dst = pl.ds(j * CH, CH)
                descs.append(pltpu.async_copy(
                    pos_hbm.at[sidx_v.at[row]], bs_v.at[dst], sem))
                descs.append(pltpu.async_copy(
                    pos_hbm.at[ridx_v.at[row]], br_v.at[dst], sem))
                descs.append(pltpu.async_copy(
                    h_hbm.at[sidx_v.at[row]], bh_v.at[dst], sem))
            for d in descs:
                d.wait()
            pltpu.sync_copy(bs_v, ps_out.at[pl.ds(base, SUPE)])
            pltpu.sync_copy(br_v, pr_out.at[pl.ds(base, SUPE)])
            pltpu.sync_copy(bh_v, hs_out.at[pl.ds(base, SUPE)])
            return carry

        lax.fori_loop(0, NSUP, sup_body, 0)

    return k(pos4, h0, send2d, recv2d)


def _sc_gather1(h, send2d):
    @functools.partial(
        pl.kernel,
        out_type=jax.ShapeDtypeStruct((E_PAD, C), _f32),
        mesh=_mesh(),
        compiler_params=pltpu.CompilerParams(use_tc_tiling_on_sc=False),
        scratch_types=[
            pltpu.VMEM((NCH, CH), jnp.int32),
            pltpu.VMEM((SUPE, C), _f32),
            pltpu.SemaphoreType.DMA,
        ],
    )
    def k(h_hbm, sidx_hbm, hs_out, sidx_v, bh_v, sem):
        wid = lax.axis_index("c") * 16 + lax.axis_index("s")
        pltpu.sync_copy(sidx_hbm.at[pl.ds(wid * NCH, NCH)], sidx_v)

        def sup_body(si, carry):
            base = wid * EPW + si * SUPE
            descs = []
            for j in range(SUP):
                row = si * SUP + j
                descs.append(pltpu.async_copy(
                    h_hbm.at[sidx_v.at[row]], bh_v.at[pl.ds(j * CH, CH)], sem))
            for d in descs:
                d.wait()
            pltpu.sync_copy(bh_v, hs_out.at[pl.ds(base, SUPE)])
            return carry

        lax.fori_loop(0, NSUP, sup_body, 0)

    return k(h, send2d)


def _sc_scatter(z, recv2d, zeros_tab):
    @functools.partial(
        pl.kernel,
        out_type=jax.ShapeDtypeStruct((2, N_PAD, C), _f32),
        mesh=_mesh(),
        compiler_params=pltpu.CompilerParams(use_tc_tiling_on_sc=False),
        scratch_types=[
            pltpu.VMEM((NCH, CH), jnp.int32),
            pltpu.VMEM((SUPE, C), _f32),
            pltpu.VMEM_SHARED((N_PAD, C), _f32),
            pltpu.SemaphoreType.DMA,
        ],
    )
    def k(z_hbm, ridx_hbm, zz_hbm, out_hbm, ridx_v, zb_v, table, sem):
        cid = lax.axis_index("c")
        sid = lax.axis_index("s")
        wid = cid * 16 + sid
        rows = N_PAD // 16
        pltpu.sync_copy(zz_hbm.at[pl.ds(sid * rows, rows)],
                        table.at[pl.ds(sid * rows, rows)])
        pltpu.sync_copy(ridx_hbm.at[pl.ds(wid * NCH, NCH)], ridx_v)
        plsc.subcore_barrier()

        def sup_body(si, carry):
            base = wid * EPW + si * SUPE
            pltpu.sync_copy(z_hbm.at[pl.ds(base, SUPE)], zb_v)
            for j in range(SUP):
                row = si * SUP + j
                pltpu.sync_copy(zb_v.at[pl.ds(j * CH, CH)],
                                table.at[ridx_v.at[row]], add=True)
            return carry

        lax.fori_loop(0, NSUP, sup_body, 0)
        plsc.subcore_barrier()
        pltpu.sync_copy(table.at[pl.ds(sid * rows, rows)],
                        out_hbm.at[cid, pl.ds(sid * rows, rows)])

    return k(z, recv2d, zeros_tab)


def kernel(positions, node_attrs, edge_index, shifts, batch, ptr, W_emb,
           W_at, b_at, R1, R2, R3, R4, W_up, W_msg, W_sc, U, Wr0, Wr1a,
           Wr1b):
    f32 = _f32
    sender = edge_index[0].astype(jnp.int32)
    receiver = edge_index[1].astype(jnp.int32)
    send2d = jnp.pad(sender, (0, E_PAD - E)).reshape(NW * NCH, CH)
    recv2d = jnp.pad(receiver, (0, E_PAD - E)).reshape(NW * NCH, CH)
    pos4 = jnp.pad(positions.astype(f32), ((0, N_PAD - N), (0, 1)))
    sh4 = jnp.pad(shifts.astype(f32), ((0, E_PAD - E), (0, 1)))
    na_p = jnp.pad(node_attrs.astype(f32), ((0, N_PAD - N), (0, 0)))
    bat_p = jnp.pad(batch.astype(jnp.int32), (0, N_PAD - N),
                    constant_values=NG).reshape(N_PAD, 1)
    zeros_tab = jnp.zeros((N_PAD, C), f32)
    b_at2 = b_at.astype(f32).reshape(1, 1)

    nf0, h0, e0 = _tc_embed(na_p, bat_p, W_emb.astype(f32),
                            W_at.astype(f32), b_at2, W_up[0].astype(f32))

    ps, pr, hs0 = _sc_gather3(pos4, h0, send2d, recv2d)

    z0, ysph, ef = _tc_edge0(ps, pr, sh4, hs0, R1[0], R2[0], R3[0], R4[0],
                             W_msg[0])

    mp0 = _sc_scatter(z0, recv2d, zeros_tab)

    nf1, h1, e1 = _tc_update0(mp0, na_p, nf0, bat_p, W_sc[0], U[0], Wr0,
                              W_up[1])

    hs1 = _sc_gather1(h1, send2d)

    z1 = _tc_edge1(ysph, ef, hs1, R1[1], R2[1], R3[1], R4[1], W_msg[1])

    mp1 = _sc_scatter(z1, recv2d, zeros_tab)

    energies = _tc_update1(mp1, na_p, nf1, bat_p, W_sc[1], U[1], Wr1a, Wr1b,
                           e0, e1)

    return energies.reshape(NG)


# speedup vs baseline: 2.0754x; 1.7053x over previous
import functools
import math

import jax
import jax.numpy as jnp
from jax import lax
from jax.experimental import pallas as pl
from jax.experimental.pallas import tpu as pltpu
from jax.experimental.pallas import tpu_sc as plsc

N = 10000
E = 160000
NE = 10
C = 32
NB = 8
NG = 16
SH = 16
RMAX = 5.0
AVG = 16.0
EPS = 1e-08

N_PAD = 10240
E_PAD = 163840
NW = 32
EPW = E_PAD // NW
CH = 128
NCH = EPW // CH
SUP = 8
SUPE = SUP * CH
NSUP = NCH // SUP
EB = 1024
EGRID = E_PAD // EB

_S3 = math.sqrt(3.0)
_S5 = math.sqrt(5.0)
_S7 = math.sqrt(7.0)
_S15 = math.sqrt(15.0)
_S42 = math.sqrt(42.0)
_S70 = math.sqrt(70.0)
_S105 = math.sqrt(105.0)

@functools.cache
def _mesh():
    return plsc.VectorSubcoreMesh(core_axis_name="c", subcore_axis_name="s",
                                  num_cores=2, num_subcores=16)


_f32 = jnp.float32


NBLK = 1024
NGRID = N_PAD // NBLK


def _onehot_energy(bat, ne):
    oh = (bat == lax.broadcasted_iota(jnp.int32, (NBLK, NG), 1)).astype(_f32)
    return jnp.sum(oh * ne, axis=0, keepdims=True)


def _acc_energy(e_ref, contrib):
    @pl.when(pl.program_id(0) == 0)
    def _():
        e_ref[...] = jnp.zeros_like(e_ref)

    e_ref[...] += contrib


def _embed_body(na_ref, bat_ref, wemb_ref, wat_ref, bat_b_ref, wup_ref,
                nf_ref, h_ref, e_ref):
    na = na_ref[...]
    nf = jnp.dot(na, wemb_ref[...], preferred_element_type=_f32)
    nf_ref[...] = nf
    h_ref[...] = jnp.dot(nf, wup_ref[...], preferred_element_type=_f32)
    ne = jnp.dot(na, wat_ref[...], preferred_element_type=_f32) + bat_b_ref[0, 0]
    _acc_energy(e_ref, _onehot_energy(bat_ref[...], ne))


def _nb_spec(w):
    return pl.BlockSpec((NBLK, w), lambda i: (i, 0))


def _w_spec(*dims):
    nd = len(dims)
    return pl.BlockSpec(dims, lambda i: (0,) * nd)


def _tc_embed(na_p, bat_p, W_emb, W_at, b_at2, Wup0):
    return pl.pallas_call(
        _embed_body,
        grid=(NGRID,),
        in_specs=[
            _nb_spec(NE), _nb_spec(1), _w_spec(NE, C), _w_spec(NE, 1),
            _w_spec(1, 1), _w_spec(C, C),
        ],
        out_specs=[_nb_spec(C), _nb_spec(C), _w_spec(1, NG)],
        out_shape=[
            jax.ShapeDtypeStruct((N_PAD, C), _f32),
            jax.ShapeDtypeStruct((N_PAD, C), _f32),
            jax.ShapeDtypeStruct((1, NG), _f32),
        ],
    )(na_p, bat_p, W_emb, W_at, b_at2, Wup0)


def _radial_mlp(ef, r1, r2, r3, r4):
    a = jax.nn.silu(jnp.dot(ef, r1, preferred_element_type=_f32))
    a = jax.nn.silu(jnp.dot(a, r2, preferred_element_type=_f32))
    a = jax.nn.silu(jnp.dot(a, r3, preferred_element_type=_f32))
    return jnp.dot(a, r4, preferred_element_type=_f32)


def _edge_msg(he, ycols, wmr):
    z = jnp.dot(he * ycols[0], wmr[0], preferred_element_type=_f32)
    for s in range(1, SH):
        z = z + jnp.dot(he * ycols[s], wmr[s], preferred_element_type=_f32)
    return z


def _edge0_body(ps_ref, pr_ref, sh_ref, hs_ref, r1_ref, r2_ref, r3_ref,
                r4_ref, wm_ref, z_ref, y_ref, ef_ref):
    i = pl.program_id(0)
    vec = pr_ref[...] - ps_ref[...] + sh_ref[...]
    ln = jnp.sqrt(jnp.sum(vec * vec, axis=1, keepdims=True))
    lc = jnp.maximum(ln, EPS)
    u = vec / lc
    x = u[:, 0:1]
    y = u[:, 1:2]
    z = u[:, 2:3]
    x2 = x * x
    y2 = y * y
    z2 = z * z
    ycols = [
        jnp.ones_like(x),
        _S3 * x, _S3 * y, _S3 * z,
        _S15 * x * y, _S15 * y * z, 0.5 * _S5 * (3.0 * z2 - 1.0),
        _S15 * x * z, 0.5 * _S15 * (x2 - y2),
        0.25 * _S70 * y * (3.0 * x2 - y2), _S105 * x * y * z,
        0.25 * _S42 * y * (5.0 * z2 - 1.0),
        0.5 * _S7 * z * (5.0 * z2 - 3.0),
        0.25 * _S42 * x * (5.0 * z2 - 1.0),
        0.5 * _S105 * z * (x2 - y2),
        0.25 * _S70 * x * (x2 - 3.0 * y2),
    ]
    y_ref[...] = jnp.concatenate(ycols, axis=1)
    nvec = lax.broadcasted_iota(jnp.int32, (1, NB), 1).astype(_f32) + 1.0
    bes = math.sqrt(2.0 / RMAX) * jnp.sin(nvec * (math.pi / RMAX) * lc) / lc
    xx = lc * (1.0 / RMAX)
    xp2 = xx * xx
    xp4 = xp2 * xp2
    xp5 = xp4 * xx
    env = 1.0 - 21.0 * xp5 + 35.0 * xp5 * xx - 15.0 * xp5 * xp2
    env = env * (xx < 1.0).astype(_f32)
    ef = bes * env
    eid = i * EB + lax.broadcasted_iota(jnp.int32, (EB, 1), 0)
    ef = jnp.where(eid < E, ef, 0.0)
    ef_ref[...] = ef
    r = _radial_mlp(ef, r1_ref[...], r2_ref[...], r3_ref[...], r4_ref[...])
    he = hs_ref[...] * r
    z_ref[...] = _edge_msg(he, ycols, wm_ref[...])


def _tc_edge0(ps, pr, sh4, hs, r1, r2, r3, r4, wm):
    eb_spec = lambda w: pl.BlockSpec((EB, w), lambda i: (i, 0))
    wspec = lambda a, b: pl.BlockSpec((a, b), lambda i: (0, 0))
    wspec3 = lambda a, b, c: pl.BlockSpec((a, b, c), lambda i: (0, 0, 0))
    return pl.pallas_call(
        _edge0_body,
        grid=(EGRID,),
        in_specs=[
            eb_spec(4), eb_spec(4), eb_spec(4), eb_spec(C),
            wspec(NB, 64), wspec(64, 64), wspec(64, 64), wspec(64, C),
            wspec3(SH, C, C),
        ],
        out_specs=[eb_spec(C), eb_spec(SH), eb_spec(NB)],
        out_shape=[
            jax.ShapeDtypeStruct((E_PAD, C), _f32),
            jax.ShapeDtypeStruct((E_PAD, SH), _f32),
            jax.ShapeDtypeStruct((E_PAD, NB), _f32),
        ],
    )(ps, pr, sh4, hs, r1, r2, r3, r4, wm)


def _edge1_body(y_ref, ef_ref, hs_ref, r1_ref, r2_ref, r3_ref, r4_ref,
                wm_ref, z_ref):
    r = _radial_mlp(ef_ref[...], r1_ref[...], r2_ref[...], r3_ref[...],
                    r4_ref[...])
    he = hs_ref[...] * r
    ysph = y_ref[...]
    ycols = [ysph[:, s:s + 1] for s in range(SH)]
    z_ref[...] = _edge_msg(he, ycols, wm_ref[...])


def _tc_edge1(ysph, ef, hs, r1, r2, r3, r4, wm):
    eb_spec = lambda w: pl.BlockSpec((EB, w), lambda i: (i, 0))
    wspec = lambda a, b: pl.BlockSpec((a, b), lambda i: (0, 0))
    wspec3 = lambda a, b, c: pl.BlockSpec((a, b, c), lambda i: (0, 0, 0))
    return pl.pallas_call(
        _edge1_body,
        grid=(EGRID,),
        in_specs=[
            eb_spec(SH), eb_spec(NB), eb_spec(C),
            wspec(NB, 64), wspec(64, 64), wspec(64, 64), wspec(64, C),
            wspec3(SH, C, C),
        ],
        out_specs=[eb_spec(C)],
        out_shape=[jax.ShapeDtypeStruct((E_PAD, C), _f32)],
    )(ysph, ef, hs, r1, r2, r3, r4, wm)[0]


def _node_update(mp, na, nf, wsc, uu):
    m = (mp[0] + mp[1]) * (1.0 / AVG)
    sc = jnp.zeros((NBLK, C), _f32)
    for k in range(NE):
        sc = sc + na[:, k:k + 1] * jnp.dot(nf, wsc[k],
                                           preferred_element_type=_f32)
    m2 = m * m
    m3 = m2 * m
    return (sc
            + jnp.dot(m, uu[0], preferred_element_type=_f32)
            + jnp.dot(m2, uu[1], preferred_element_type=_f32)
            + jnp.dot(m3, uu[2], preferred_element_type=_f32))


def _update0_body(mp_ref, na_ref, nf_ref, bat_ref, wsc_ref, u_ref, wr0_ref,
                  wup_ref, nfo_ref, h_ref, e_ref):
    nfo = _node_update(mp_ref[...], na_ref[...], nf_ref[...], wsc_ref[...],
                       u_ref[...])
    nfo_ref[...] = nfo
    h_ref[...] = jnp.dot(nfo, wup_ref[...], preferred_element_type=_f32)
    ne = jnp.dot(nfo, wr0_ref[...], preferred_element_type=_f32)
    _acc_energy(e_ref, _onehot_energy(bat_ref[...], ne))


def _mp_spec():
    return pl.BlockSpec((2, NBLK, C), lambda i: (0, i, 0))


def _tc_update0(mp, na_p, nf, bat_p, wsc, uu, wr0, wup1):
    return pl.pallas_call(
        _update0_body,
        grid=(NGRID,),
        in_specs=[
            _mp_spec(), _nb_spec(NE), _nb_spec(C), _nb_spec(1),
            _w_spec(NE, C, C), _w_spec(3, C, C), _w_spec(C, 1),
            _w_spec(C, C),
        ],
        out_specs=[_nb_spec(C), _nb_spec(C), _w_spec(1, NG)],
        out_shape=[
            jax.ShapeDtypeStruct((N_PAD, C), _f32),
            jax.ShapeDtypeStruct((N_PAD, C), _f32),
            jax.ShapeDtypeStruct((1, NG), _f32),
        ],
    )(mp, na_p, nf, bat_p, wsc, uu, wr0, wup1)


def _update1_body(mp_ref, na_ref, nf_ref, bat_ref, wsc_ref, u_ref, wr1a_ref,
                  wr1b_ref, e0_ref, e1_ref, e_ref):
    nfo = _node_update(mp_ref[...], na_ref[...], nf_ref[...], wsc_ref[...],
                       u_ref[...])
    hid = jax.nn.silu(jnp.dot(nfo, wr1a_ref[...], preferred_element_type=_f32))
    ne = jnp.dot(hid, wr1b_ref[...], preferred_element_type=_f32)
    contrib = _onehot_energy(bat_ref[...], ne)

    @pl.when(pl.program_id(0) == 0)
    def _():
        e_ref[...] = e0_ref[...] + e1_ref[...]

    e_ref[...] += contrib


def _tc_update1(mp, na_p, nf, bat_p, wsc, uu, wr1a, wr1b, e0, e1):
    return pl.pallas_call(
        _update1_body,
        grid=(NGRID,),
        in_specs=[
            _mp_spec(), _nb_spec(NE), _nb_spec(C), _nb_spec(1),
            _w_spec(NE, C, C), _w_spec(3, C, C), _w_spec(C, SH),
            _w_spec(SH, 1), _w_spec(1, NG), _w_spec(1, NG),
        ],
        out_specs=_w_spec(1, NG),
        out_shape=jax.ShapeDtypeStruct((1, NG), _f32),
    )(mp, na_p, nf, bat_p, wsc, uu, wr1a, wr1b, e0, e1)


def _sc_gather3(pos4, h0, send2d, recv2d):
    @functools.partial(
        pl.kernel,
        out_type=[
            jax.ShapeDtypeStruct((E_PAD, 4), _f32),
            jax.ShapeDtypeStruct((E_PAD, 4), _f32),
            jax.ShapeDtypeStruct((E_PAD, C), _f32),
        ],
        mesh=_mesh(),
        compiler_params=pltpu.CompilerParams(use_tc_tiling_on_sc=False),
        scratch_types=[
            pltpu.VMEM((NCH, CH), jnp.int32),
            pltpu.VMEM((NCH, CH), jnp.int32),
            pltpu.VMEM((SUPE, 4), _f32),
            pltpu.VMEM((SUPE, 4), _f32),
            pltpu.VMEM((SUPE, C), _f32),
            pltpu.SemaphoreType.DMA,
        ],
    )
    def k(pos_hbm, h_hbm, sidx_hbm, ridx_hbm, ps_out, pr_out, hs_out,
          sidx_v, ridx_v, bs_v, br_v, bh_v, sem):
        wid = lax.axis_index("c") * 16 + lax.axis_index("s")
        pltpu.sync_copy(sidx_hbm.at[pl.ds(wid * NCH, NCH)], sidx_v)
        pltpu.sync_copy(ridx_hbm.at[pl.ds(wid * NCH, NCH)], ridx_v)

        def sup_body(si, carry):
            base = wid * EPW + si * SUPE
            descs = []
            for j in range(SUP):
                row = si * SUP + j
                dst = pl.ds(j * CH, CH)
                descs.append(pltpu.async_copy(
                    pos_hbm.at[sidx_v.at[row]], bs_v.at[dst], sem))
                descs.append(pltpu.async_copy(
                    pos_hbm.at[ridx_v.at[row]], br_v.at[dst], sem))
                descs.append(pltpu.async_copy(
                    h_hbm.at[sidx_v.at[row]], bh_v.at[dst], sem))
            for d in descs:
                d.wait()
            pltpu.sync_copy(bs_v, ps_out.at[pl.ds(base, SUPE)])
            pltpu.sync_copy(br_v, pr_out.at[pl.ds(base, SUPE)])
            pltpu.sync_copy(bh_v, hs_out.at[pl.ds(base, SUPE)])
            return carry

        lax.fori_loop(0, NSUP, sup_body, 0)

    return k(pos4, h0, send2d, recv2d)


def _sc_gather1(h, send2d):
    @functools.partial(
        pl.kernel,
        out_type=jax.ShapeDtypeStruct((E_PAD, C), _f32),
        mesh=_mesh(),
        compiler_params=pltpu.CompilerParams(use_tc_tiling_on_sc=False),
        scratch_types=[
            pltpu.VMEM((NCH, CH), jnp.int32),
            pltpu.VMEM((SUPE, C), _f32),
            pltpu.SemaphoreType.DMA,
        ],
    )
    def k(h_hbm, sidx_hbm, hs_out, sidx_v, bh_v, sem):
        wid = lax.axis_index("c") * 16 + lax.axis_index("s")
        pltpu.sync_copy(sidx_hbm.at[pl.ds(wid * NCH, NCH)], sidx_v)

        def sup_body(si, carry):
            base = wid * EPW + si * SUPE
            descs = []
            for j in range(SUP):
                row = si * SUP + j
                descs.append(pltpu.async_copy(
                    h_hbm.at[sidx_v.at[row]], bh_v.at[pl.ds(j * CH, CH)], sem))
            for d in descs:
                d.wait()
            pltpu.sync_copy(bh_v, hs_out.at[pl.ds(base, SUPE)])
            return carry

        lax.fori_loop(0, NSUP, sup_body, 0)

    return k(h, send2d)


def _sc_scatter(z, recv2d, zeros_tab):
    @functools.partial(
        pl.kernel,
        out_type=jax.ShapeDtypeStruct((2, N_PAD, C), _f32),
        mesh=_mesh(),
        compiler_params=pltpu.CompilerParams(use_tc_tiling_on_sc=False),
        scratch_types=[
            pltpu.VMEM((NCH, CH), jnp.int32),
            pltpu.VMEM((SUPE, C), _f32),
            pltpu.VMEM_SHARED((N_PAD, C), _f32),
            pltpu.SemaphoreType.DMA,
        ],
    )
    def k(z_hbm, ridx_hbm, zz_hbm, out_hbm, ridx_v, zb_v, table, sem):
        cid = lax.axis_index("c")
        sid = lax.axis_index("s")
        wid = cid * 16 + sid
        rows = N_PAD // 16
        pltpu.sync_copy(zz_hbm.at[pl.ds(sid * rows, rows)],
                        table.at[pl.ds(sid * rows, rows)])
        pltpu.sync_copy(ridx_hbm.at[pl.ds(wid * NCH, NCH)], ridx_v)
        plsc.subcore_barrier()

        def sup_body(si, carry):
            base = wid * EPW + si * SUPE
            pltpu.sync_copy(z_hbm.at[pl.ds(base, SUPE)], zb_v)
            for j in range(SUP):
                row = si * SUP + j
                pltpu.sync_copy(zb_v.at[pl.ds(j * CH, CH)],
                                table.at[ridx_v.at[row]], add=True)
            return carry

        lax.fori_loop(0, NSUP, sup_body, 0)
        plsc.subcore_barrier()
        pltpu.sync_copy(table.at[pl.ds(sid * rows, rows)],
                        out_hbm.at[cid, pl.ds(sid * rows, rows)])

    return k(z, recv2d, zeros_tab)


def kernel(positions, node_attrs, edge_index, shifts, batch, ptr, W_emb,
           W_at, b_at, R1, R2, R3, R4, W_up, W_msg, W_sc, U, Wr0, Wr1a,
           Wr1b):
    f32 = _f32
    sender = edge_index[0].astype(jnp.int32)
    receiver = edge_index[1].astype(jnp.int32)
    send2d = jnp.pad(sender, (0, E_PAD - E)).reshape(NW * NCH, CH)
    recv2d = jnp.pad(receiver, (0, E_PAD - E)).reshape(NW * NCH, CH)
    pos4 = jnp.pad(positions.astype(f32), ((0, N_PAD - N), (0, 1)))
    sh4 = jnp.pad(shifts.astype(f32), ((0, E_PAD - E), (0, 1)))
    na_p = jnp.pad(node_attrs.astype(f32), ((0, N_PAD - N), (0, 0)))
    bat_p = jnp.pad(batch.astype(jnp.int32), (0, N_PAD - N),
                    constant_values=NG).reshape(N_PAD, 1)
    zeros_tab = jnp.zeros((N_PAD, C), f32)
    b_at2 = b_at.astype(f32).reshape(1, 1)

    nf0, h0, e0 = _tc_embed(na_p, bat_p, W_emb.astype(f32),
                            W_at.astype(f32), b_at2, W_up[0].astype(f32))

    ps, pr, hs0 = _sc_gather3(pos4, h0, send2d, recv2d)

    z0, ysph, ef = _tc_edge0(ps, pr, sh4, hs0, R1[0], R2[0], R3[0], R4[0],
                             W_msg[0].reshape(C, SH, C).transpose(1, 0, 2))

    mp0 = _sc_scatter(z0, recv2d, zeros_tab)

    nf1, h1, e1 = _tc_update0(mp0, na_p, nf0, bat_p, W_sc[0], U[0], Wr0,
                              W_up[1])

    hs1 = _sc_gather1(h1, send2d)

    z1 = _tc_edge1(ysph, ef, hs1, R1[1], R2[1], R3[1], R4[1],
                   W_msg[1].reshape(C, SH, C).transpose(1, 0, 2))

    mp1 = _sc_scatter(z1, recv2d, zeros_tab)

    energies = _tc_update1(mp1, na_p, nf1, bat_p, W_sc[1], U[1], Wr1a, Wr1b,
                           e0, e1)

    return energies.reshape(NG)


# speedup vs baseline: 3.9199x; 1.8888x over previous
import functools
import math

import jax
import jax.numpy as jnp
from jax import lax
from jax.experimental import pallas as pl
from jax.experimental.pallas import tpu as pltpu
from jax.experimental.pallas import tpu_sc as plsc

N = 10000
E = 160000
NE = 10
C = 32
NB = 8
NG = 16
SH = 16
RMAX = 5.0
AVG = 16.0
EPS = 1e-08

N_PAD = 10240
E_PAD = 163840
NW = 32
EPW = E_PAD // NW
CH = 128
NCH = EPW // CH
SUP = 8
SUPE = SUP * CH
NSUP = NCH // SUP
EB = 1024
EGRID = E_PAD // EB

_S3 = math.sqrt(3.0)
_S5 = math.sqrt(5.0)
_S7 = math.sqrt(7.0)
_S15 = math.sqrt(15.0)
_S42 = math.sqrt(42.0)
_S70 = math.sqrt(70.0)
_S105 = math.sqrt(105.0)

@functools.cache
def _mesh():
    return plsc.VectorSubcoreMesh(core_axis_name="c", subcore_axis_name="s",
                                  num_cores=2, num_subcores=16)


_f32 = jnp.float32


NBLK = 1024
NGRID = N_PAD // NBLK


def _onehot_energy(bat, ne):
    oh = (bat == lax.broadcasted_iota(jnp.int32, (NBLK, NG), 1)).astype(_f32)
    return jnp.sum(oh * ne, axis=0, keepdims=True)


def _acc_energy(e_ref, contrib):
    @pl.when(pl.program_id(0) == 0)
    def _():
        e_ref[...] = jnp.zeros_like(e_ref)

    e_ref[...] += contrib


def _embed_body(na_ref, bat_ref, wemb_ref, wat_ref, bat_b_ref, wup_ref,
                nf_ref, h_ref, e_ref):
    na = na_ref[...]
    nf = jnp.dot(na, wemb_ref[...], preferred_element_type=_f32)
    nf_ref[...] = nf
    h_ref[...] = jnp.dot(nf, wup_ref[...], preferred_element_type=_f32)
    ne = jnp.dot(na, wat_ref[...], preferred_element_type=_f32) + bat_b_ref[0, 0]
    _acc_energy(e_ref, _onehot_energy(bat_ref[...], ne))


def _nb_spec(w):
    return pl.BlockSpec((NBLK, w), lambda i: (i, 0))


def _w_spec(*dims):
    nd = len(dims)
    return pl.BlockSpec(dims, lambda i: (0,) * nd)


def _tc_embed(na_p, bat_p, W_emb, W_at, b_at2, Wup0):
    return pl.pallas_call(
        _embed_body,
        grid=(NGRID,),
        in_specs=[
            _nb_spec(NE), _nb_spec(1), _w_spec(NE, C), _w_spec(NE, 1),
            _w_spec(1, 1), _w_spec(C, C),
        ],
        out_specs=[_nb_spec(C), _nb_spec(C), _w_spec(1, NG)],
        out_shape=[
            jax.ShapeDtypeStruct((N_PAD, C), _f32),
            jax.ShapeDtypeStruct((N_PAD, C), _f32),
            jax.ShapeDtypeStruct((1, NG), _f32),
        ],
    )(na_p, bat_p, W_emb, W_at, b_at2, Wup0)


def _radial_mlp(ef, r1, r2, r3, r4):
    a = jax.nn.silu(jnp.dot(ef, r1, preferred_element_type=_f32))
    a = jax.nn.silu(jnp.dot(a, r2, preferred_element_type=_f32))
    a = jax.nn.silu(jnp.dot(a, r3, preferred_element_type=_f32))
    return jnp.dot(a, r4, preferred_element_type=_f32)


def _edge_msg_t(het, yrows, wmrt):
    zt = jnp.dot(wmrt[0], het * yrows[0], preferred_element_type=_f32)
    for s in range(1, SH):
        zt = zt + jnp.dot(wmrt[s], het * yrows[s],
                          preferred_element_type=_f32)
    return zt


def _radial_mlp_t(eft, r1t, r2t, r3t, r4t):
    a = jax.nn.silu(jnp.dot(r1t, eft, preferred_element_type=_f32))
    a = jax.nn.silu(jnp.dot(r2t, a, preferred_element_type=_f32))
    a = jax.nn.silu(jnp.dot(r3t, a, preferred_element_type=_f32))
    return jnp.dot(r4t, a, preferred_element_type=_f32)


def _edge0_body(ps_ref, pr_ref, sh_ref, hs_ref, r1_ref, r2_ref, r3_ref,
                r4_ref, wm_ref, z_ref, y_ref, ef_ref):
    i = pl.program_id(0)
    vec = pr_ref[...] - ps_ref[...] + sh_ref[...]
    vt = jnp.transpose(vec)
    x = vt[0:1, :]
    y = vt[1:2, :]
    z = vt[2:3, :]
    lc = jnp.maximum(jnp.sqrt(x * x + y * y + z * z), EPS)
    inv = 1.0 / lc
    x = x * inv
    y = y * inv
    z = z * inv
    x2 = x * x
    y2 = y * y
    z2 = z * z
    yrows = [
        jnp.ones_like(x),
        _S3 * x, _S3 * y, _S3 * z,
        _S15 * x * y, _S15 * y * z, 0.5 * _S5 * (3.0 * z2 - 1.0),
        _S15 * x * z, 0.5 * _S15 * (x2 - y2),
        0.25 * _S70 * y * (3.0 * x2 - y2), _S105 * x * y * z,
        0.25 * _S42 * y * (5.0 * z2 - 1.0),
        0.5 * _S7 * z * (5.0 * z2 - 3.0),
        0.25 * _S42 * x * (5.0 * z2 - 1.0),
        0.5 * _S105 * z * (x2 - y2),
        0.25 * _S70 * x * (x2 - 3.0 * y2),
    ]
    y_ref[...] = jnp.concatenate(yrows, axis=0)
    nvec = lax.broadcasted_iota(jnp.int32, (NB, 1), 0).astype(_f32) + 1.0
    bes = math.sqrt(2.0 / RMAX) * jnp.sin(nvec * ((math.pi / RMAX) * lc)) * inv
    xx = lc * (1.0 / RMAX)
    xp2 = xx * xx
    xp4 = xp2 * xp2
    xp5 = xp4 * xx
    env = 1.0 - 21.0 * xp5 + 35.0 * xp5 * xx - 15.0 * xp5 * xp2
    env = env * (xx < 1.0).astype(_f32)
    eid = i * EB + lax.broadcasted_iota(jnp.int32, (1, EB), 1)
    env = jnp.where(eid < E, env, 0.0)
    eft = bes * env
    ef_ref[...] = eft
    rt = _radial_mlp_t(eft, r1_ref[...], r2_ref[...], r3_ref[...],
                       r4_ref[...])
    het = jnp.transpose(hs_ref[...]) * rt
    z_ref[...] = jnp.transpose(_edge_msg_t(het, yrows, wm_ref[...]))


def _eb_spec(w):
    return pl.BlockSpec((EB, w), lambda i: (i, 0))


def _et_spec(h):
    return pl.BlockSpec((h, EB), lambda i: (0, i))


def _tc_edge0(ps, pr, sh4, hs, r1t, r2t, r3t, r4t, wmt):
    return pl.pallas_call(
        _edge0_body,
        grid=(EGRID,),
        in_specs=[
            _eb_spec(4), _eb_spec(4), _eb_spec(4), _eb_spec(C),
            _w_spec(64, NB), _w_spec(64, 64), _w_spec(64, 64), _w_spec(C, 64),
            _w_spec(SH, C, C),
        ],
        out_specs=[_eb_spec(C), _et_spec(SH), _et_spec(NB)],
        out_shape=[
            jax.ShapeDtypeStruct((E_PAD, C), _f32),
            jax.ShapeDtypeStruct((SH, E_PAD), _f32),
            jax.ShapeDtypeStruct((NB, E_PAD), _f32),
        ],
    )(ps, pr, sh4, hs, r1t, r2t, r3t, r4t, wmt)


def _edge1_body(y_ref, ef_ref, hs_ref, r1_ref, r2_ref, r3_ref, r4_ref,
                wm_ref, z_ref):
    rt = _radial_mlp_t(ef_ref[...], r1_ref[...], r2_ref[...], r3_ref[...],
                       r4_ref[...])
    het = jnp.transpose(hs_ref[...]) * rt
    yt = y_ref[...]
    yrows = [yt[s:s + 1, :] for s in range(SH)]
    z_ref[...] = jnp.transpose(_edge_msg_t(het, yrows, wm_ref[...]))


def _tc_edge1(ysph, ef, hs, r1t, r2t, r3t, r4t, wmt):
    return pl.pallas_call(
        _edge1_body,
        grid=(EGRID,),
        in_specs=[
            _et_spec(SH), _et_spec(NB), _eb_spec(C),
            _w_spec(64, NB), _w_spec(64, 64), _w_spec(64, 64), _w_spec(C, 64),
            _w_spec(SH, C, C),
        ],
        out_specs=[_eb_spec(C)],
        out_shape=[jax.ShapeDtypeStruct((E_PAD, C), _f32)],
    )(ysph, ef, hs, r1t, r2t, r3t, r4t, wmt)[0]


def _node_update(mp, na, nf, wsc, uu):
    m = (mp[0] + mp[1]) * (1.0 / AVG)
    sc = jnp.zeros((NBLK, C), _f32)
    for k in range(NE):
        sc = sc + na[:, k:k + 1] * jnp.dot(nf, wsc[k],
                                           preferred_element_type=_f32)
    m2 = m * m
    m3 = m2 * m
    return (sc
            + jnp.dot(m, uu[0], preferred_element_type=_f32)
            + jnp.dot(m2, uu[1], preferred_element_type=_f32)
            + jnp.dot(m3, uu[2], preferred_element_type=_f32))


def _update0_body(mp_ref, na_ref, nf_ref, bat_ref, wsc_ref, u_ref, wr0_ref,
                  wup_ref, nfo_ref, h_ref, e_ref):
    nfo = _node_update(mp_ref[...], na_ref[...], nf_ref[...], wsc_ref[...],
                       u_ref[...])
    nfo_ref[...] = nfo
    h_ref[...] = jnp.dot(nfo, wup_ref[...], preferred_element_type=_f32)
    ne = jnp.dot(nfo, wr0_ref[...], preferred_element_type=_f32)
    _acc_energy(e_ref, _onehot_energy(bat_ref[...], ne))


def _mp_spec():
    return pl.BlockSpec((2, NBLK, C), lambda i: (0, i, 0))


def _tc_update0(mp, na_p, nf, bat_p, wsc, uu, wr0, wup1):
    return pl.pallas_call(
        _update0_body,
        grid=(NGRID,),
        in_specs=[
            _mp_spec(), _nb_spec(NE), _nb_spec(C), _nb_spec(1),
            _w_spec(NE, C, C), _w_spec(3, C, C), _w_spec(C, 1),
            _w_spec(C, C),
        ],
        out_specs=[_nb_spec(C), _nb_spec(C), _w_spec(1, NG)],
        out_shape=[
            jax.ShapeDtypeStruct((N_PAD, C), _f32),
            jax.ShapeDtypeStruct((N_PAD, C), _f32),
            jax.ShapeDtypeStruct((1, NG), _f32),
        ],
    )(mp, na_p, nf, bat_p, wsc, uu, wr0, wup1)


def _update1_body(mp_ref, na_ref, nf_ref, bat_ref, wsc_ref, u_ref, wr1a_ref,
                  wr1b_ref, e0_ref, e1_ref, e_ref):
    nfo = _node_update(mp_ref[...], na_ref[...], nf_ref[...], wsc_ref[...],
                       u_ref[...])
    hid = jax.nn.silu(jnp.dot(nfo, wr1a_ref[...], preferred_element_type=_f32))
    ne = jnp.dot(hid, wr1b_ref[...], preferred_element_type=_f32)
    contrib = _onehot_energy(bat_ref[...], ne)

    @pl.when(pl.program_id(0) == 0)
    def _():
        e_ref[...] = e0_ref[...] + e1_ref[...]

    e_ref[...] += contrib


def _tc_update1(mp, na_p, nf, bat_p, wsc, uu, wr1a, wr1b, e0, e1):
    return pl.pallas_call(
        _update1_body,
        grid=(NGRID,),
        in_specs=[
            _mp_spec(), _nb_spec(NE), _nb_spec(C), _nb_spec(1),
            _w_spec(NE, C, C), _w_spec(3, C, C), _w_spec(C, SH),
            _w_spec(SH, 1), _w_spec(1, NG), _w_spec(1, NG),
        ],
        out_specs=_w_spec(1, NG),
        out_shape=jax.ShapeDtypeStruct((1, NG), _f32),
    )(mp, na_p, nf, bat_p, wsc, uu, wr1a, wr1b, e0, e1)


def _sc_gather3(pos4, h0, send2d, recv2d):
    @functools.partial(
        pl.kernel,
        out_type=[
            jax.ShapeDtypeStruct((E_PAD, 4), _f32),
            jax.ShapeDtypeStruct((E_PAD, 4), _f32),
            jax.ShapeDtypeStruct((E_PAD, C), _f32),
        ],
        mesh=_mesh(),
        compiler_params=pltpu.CompilerParams(use_tc_tiling_on_sc=False),
        scratch_types=[
            pltpu.VMEM((NCH, CH), jnp.int32),
            pltpu.VMEM((NCH, CH), jnp.int32),
            pltpu.VMEM((SUPE, 4), _f32),
            pltpu.VMEM((SUPE, 4), _f32),
            pltpu.VMEM((SUPE, C), _f32),
            pltpu.SemaphoreType.DMA,
        ],
    )
    def k(pos_hbm, h_hbm, sidx_hbm, ridx_hbm, ps_out, pr_out, hs_out,
          sidx_v, ridx_v, bs_v, br_v, bh_v, sem):
        wid = lax.axis_index("c") * 16 + lax.axis_index("s")
        pltpu.sync_copy(sidx_hbm.at[pl.ds(wid * NCH, NCH)], sidx_v)
        pltpu.sync_copy(ridx_hbm.at[pl.ds(wid * NCH, NCH)], ridx_v)

        def sup_body(si, carry):
            base = wid * EPW + si * SUPE
            descs = []
            for j in range(SUP):
                row = si * SUP + j
                dst = pl.ds(j * CH, CH)
                descs.append(pltpu.async_copy(
                    pos_hbm.at[sidx_v.at[row]], bs_v.at[dst], sem))
                descs.append(pltpu.async_copy(
                    pos_hbm.at[ridx_v.at[row]], br_v.at[dst], sem))
                descs.append(pltpu.async_copy(
                    h_hbm.at[sidx_v.at[row]], bh_v.at[dst], sem))
            for d in descs:
                d.wait()
            pltpu.sync_copy(bs_v, ps_out.at[pl.ds(base, SUPE)])
            pltpu.sync_copy(br_v, pr_out.at[pl.ds(base, SUPE)])
            pltpu.sync_copy(bh_v, hs_out.at[pl.ds(base, SUPE)])
            return carry

        lax.fori_loop(0, NSUP, sup_body, 0)

    return k(pos4, h0, send2d, recv2d)


def _sc_gather1(h, send2d):
    @functools.partial(
        pl.kernel,
        out_type=jax.ShapeDtypeStruct((E_PAD, C), _f32),
        mesh=_mesh(),
        compiler_params=pltpu.CompilerParams(use_tc_tiling_on_sc=False),
        scratch_types=[
            pltpu.VMEM((NCH, CH), jnp.int32),
            pltpu.VMEM((SUPE, C), _f32),
            pltpu.SemaphoreType.DMA,
        ],
    )
    def k(h_hbm, sidx_hbm, hs_out, sidx_v, bh_v, sem):
        wid = lax.axis_index("c") * 16 + lax.axis_index("s")
        pltpu.sync_copy(sidx_hbm.at[pl.ds(wid * NCH, NCH)], sidx_v)

        def sup_body(si, carry):
            base = wid * EPW + si * SUPE
            descs = []
            for j in range(SUP):
                row = si * SUP + j
                descs.append(pltpu.async_copy(
                    h_hbm.at[sidx_v.at[row]], bh_v.at[pl.ds(j * CH, CH)], sem))
            for d in descs:
                d.wait()
            pltpu.sync_copy(bh_v, hs_out.at[pl.ds(base, SUPE)])
            return carry

        lax.fori_loop(0, NSUP, sup_body, 0)

    return k(h, send2d)


def _sc_scatter(z, recv2d, zeros_tab):
    @functools.partial(
        pl.kernel,
        out_type=jax.ShapeDtypeStruct((2, N_PAD, C), _f32),
        mesh=_mesh(),
        compiler_params=pltpu.CompilerParams(use_tc_tiling_on_sc=False),
        scratch_types=[
            pltpu.VMEM((NCH, CH), jnp.int32),
            pltpu.VMEM((SUPE, C), _f32),
            pltpu.VMEM_SHARED((N_PAD, C), _f32),
            pltpu.SemaphoreType.DMA,
        ],
    )
    def k(z_hbm, ridx_hbm, zz_hbm, out_hbm, ridx_v, zb_v, table, sem):
        cid = lax.axis_index("c")
        sid = lax.axis_index("s")
        wid = cid * 16 + sid
        rows = N_PAD // 16
        pltpu.sync_copy(zz_hbm.at[pl.ds(sid * rows, rows)],
                        table.at[pl.ds(sid * rows, rows)])
        pltpu.sync_copy(ridx_hbm.at[pl.ds(wid * NCH, NCH)], ridx_v)
        plsc.subcore_barrier()

        def sup_body(si, carry):
            base = wid * EPW + si * SUPE
            pltpu.sync_copy(z_hbm.at[pl.ds(base, SUPE)], zb_v)
            for j in range(SUP):
                row = si * SUP + j
                pltpu.sync_copy(zb_v.at[pl.ds(j * CH, CH)],
                                table.at[ridx_v.at[row]], add=True)
            return carry

        lax.fori_loop(0, NSUP, sup_body, 0)
        plsc.subcore_barrier()
        pltpu.sync_copy(table.at[pl.ds(sid * rows, rows)],
                        out_hbm.at[cid, pl.ds(sid * rows, rows)])

    return k(z, recv2d, zeros_tab)


def kernel(positions, node_attrs, edge_index, shifts, batch, ptr, W_emb,
           W_at, b_at, R1, R2, R3, R4, W_up, W_msg, W_sc, U, Wr0, Wr1a,
           Wr1b):
    f32 = _f32
    sender = edge_index[0].astype(jnp.int32)
    receiver = edge_index[1].astype(jnp.int32)
    send2d = jnp.pad(sender, (0, E_PAD - E)).reshape(NW * NCH, CH)
    recv2d = jnp.pad(receiver, (0, E_PAD - E)).reshape(NW * NCH, CH)
    pos4 = jnp.pad(positions.astype(f32), ((0, N_PAD - N), (0, 1)))
    sh4 = jnp.pad(shifts.astype(f32), ((0, E_PAD - E), (0, 1)))
    na_p = jnp.pad(node_attrs.astype(f32), ((0, N_PAD - N), (0, 0)))
    bat_p = jnp.pad(batch.astype(jnp.int32), (0, N_PAD - N),
                    constant_values=NG).reshape(N_PAD, 1)
    zeros_tab = jnp.zeros((N_PAD, C), f32)
    b_at2 = b_at.astype(f32).reshape(1, 1)

    nf0, h0, e0 = _tc_embed(na_p, bat_p, W_emb.astype(f32),
                            W_at.astype(f32), b_at2, W_up[0].astype(f32))

    ps, pr, hs0 = _sc_gather3(pos4, h0, send2d, recv2d)

    z0, ysph, ef = _tc_edge0(ps, pr, sh4, hs0, R1[0].T, R2[0].T, R3[0].T,
                             R4[0].T,
                             W_msg[0].reshape(C, SH, C).transpose(1, 2, 0))

    mp0 = _sc_scatter(z0, recv2d, zeros_tab)

    nf1, h1, e1 = _tc_update0(mp0, na_p, nf0, bat_p, W_sc[0], U[0], Wr0,
                              W_up[1])

    hs1 = _sc_gather1(h1, send2d)

    z1 = _tc_edge1(ysph, ef, hs1, R1[1].T, R2[1].T, R3[1].T, R4[1].T,
                   W_msg[1].reshape(C, SH, C).transpose(1, 2, 0))

    mp1 = _sc_scatter(z1, recv2d, zeros_tab)

    energies = _tc_update1(mp1, na_p, nf1, bat_p, W_sc[1], U[1], Wr1a, Wr1b,
                           e0, e1)

    return energies.reshape(NG)
